# baseline (speedup 1.0000x reference)
import functools
import jax
import jax.numpy as jnp
from jax import lax
from jax.experimental import pallas as pl
from jax.experimental.pallas import tpu as pltpu, tpu_sc as plsc

BATCH = 16384
PER_DEV_DIM = 1664
WORLD_SIZE = 4
OUT_DIM = WORLD_SIZE * PER_DEV_DIM

NC, NS = 2, 16
NW = NC * NS
RPW = BATCH // NW
CR = 4
NCH = RPW // CR
D = 4
R = 2

_mesh = plsc.VectorSubcoreMesh(core_axis_name="c", subcore_axis_name="s")


@functools.partial(
    pl.kernel,
    mesh=_mesh,
    out_type=jax.ShapeDtypeStruct((BATCH, OUT_DIM), jnp.float32),
    scratch_types=[
        pltpu.VMEM((D, CR, OUT_DIM), jnp.float32),
        pltpu.SemaphoreType.DMA((D,)),
        pltpu.SemaphoreType.DMA((D,)),
    ],
)
def _sc_merge(t0, t1, t2, t3, out, buf, rsem, wsem):
    wid = lax.axis_index("s") * NC + lax.axis_index("c")
    base = wid * RPW
    ins = (t0, t1, t2, t3)

    def read_copies(ch, slot):
        row = base + ch * CR
        return [
            pltpu.make_async_copy(
                ins[i].at[pl.ds(row, CR), :],
                buf.at[slot, :, pl.ds(i * PER_DEV_DIM, PER_DEV_DIM)],
                rsem.at[slot],
            )
            for i in range(WORLD_SIZE)
        ]

    def write_copy(ch, slot):
        row = base + ch * CR
        return pltpu.make_async_copy(
            buf.at[slot], out.at[pl.ds(row, CR), :], wsem.at[slot]
        )

    for b in range(R):
        for c in read_copies(b, b):
            c.start()

    @pl.loop(0, NCH, step=D)
    def _super(ch0):
        for b in range(D):
            ch = ch0 + b
            for c in read_copies(ch, b):
                c.wait()
            write_copy(ch, b).start()
            ns = (b + R) % D

            @pl.when(ch + R < NCH)
            def _prefetch():
                @pl.when(ch >= D - R)
                def _reclaim():
                    write_copy(0, ns).wait()

                for c in read_copies(ch + R, ns):
                    c.start()

    for b in range(D):
        write_copy(0, b).wait()


def kernel(tensors_0, tensors_1, tensors_2, tensors_3):
    return _sc_merge(tensors_0, tensors_1, tensors_2, tensors_3)


# speedup vs baseline: 1.1806x; 1.1806x over previous
import jax
import jax.numpy as jnp
from jax.experimental import pallas as pl

BATCH = 16384
PER_DEV_DIM = 1664
WORLD_SIZE = 4
OUT_DIM = WORLD_SIZE * PER_DEV_DIM

BR = 512


def _merge_block_kernel(t0, t1, t2, t3, out):
    out[:, 0 * PER_DEV_DIM : 1 * PER_DEV_DIM] = t0[...]
    out[:, 1 * PER_DEV_DIM : 2 * PER_DEV_DIM] = t1[...]
    out[:, 2 * PER_DEV_DIM : 3 * PER_DEV_DIM] = t2[...]
    out[:, 3 * PER_DEV_DIM : 4 * PER_DEV_DIM] = t3[...]


def kernel(tensors_0, tensors_1, tensors_2, tensors_3):
    in_spec = pl.BlockSpec((BR, PER_DEV_DIM), lambda i: (i, 0))
    out_spec = pl.BlockSpec((BR, OUT_DIM), lambda i: (i, 0))
    return pl.pallas_call(
        _merge_block_kernel,
        grid=(BATCH // BR,),
        out_shape=jax.ShapeDtypeStruct((BATCH, OUT_DIM), jnp.float32),
        in_specs=[in_spec] * WORLD_SIZE,
        out_specs=out_spec,
    )(tensors_0, tensors_1, tensors_2, tensors_3)
